# baseline (speedup 1.0000x reference)
import functools

import jax
import jax.numpy as jnp
from jax import lax
from jax.experimental import pallas as pl
from jax.experimental.pallas import tpu as pltpu
from jax.experimental.pallas import tpu_sc as plsc

N = 10000
F = 128
D = 128
B = 4096

ROW_BLK = 400


def _encode_body(feat_ref, adj_ref, w_ref, b_ref, x_ref, support_ref):
    @pl.when(pl.program_id(0) == 0)
    def _():
        s = jnp.dot(
            feat_ref[...].astype(jnp.bfloat16),
            w_ref[...].astype(jnp.bfloat16),
            preferred_element_type=jnp.float32,
        )
        support_ref[...] = s.astype(jnp.bfloat16)

    acc = jnp.dot(
        adj_ref[...].astype(jnp.bfloat16),
        support_ref[...],
        preferred_element_type=jnp.float32,
    )
    x_ref[...] = jnp.tanh(acc + b_ref[...])


def _encode(feature, adj, W, b2d):
    grid = (N // ROW_BLK,)
    return pl.pallas_call(
        _encode_body,
        grid=grid,
        in_specs=[
            pl.BlockSpec((N, F), lambda i: (0, 0)),
            pl.BlockSpec((ROW_BLK, N), lambda i: (i, 0)),
            pl.BlockSpec((F, D), lambda i: (0, 0)),
            pl.BlockSpec((1, D), lambda i: (0, 0)),
        ],
        out_specs=pl.BlockSpec((ROW_BLK, D), lambda i: (i, 0)),
        out_shape=jax.ShapeDtypeStruct((N, D), jnp.float32),
        scratch_shapes=[pltpu.VMEM((N, D), jnp.bfloat16)],
    )(feature, adj, W, b2d)


_NC = 2
_NS = 16
_NW = _NC * _NS
_BPW = B // _NW


def _decode_body(x_hbm, u_hbm, ip_hbm, in_hbm, pp_hbm, pn_hbm,
                 outp_hbm, outn_hbm,
                 iu, iip, iin, ipp, ipn,
                 ru, rip, rin, rpp, rpn,
                 op_v, on_v, sem):
    wid = lax.axis_index("s") * _NC + lax.axis_index("c")
    base = wid * _BPW
    pltpu.sync_copy(u_hbm.at[pl.ds(base, _BPW)], iu)
    pltpu.sync_copy(ip_hbm.at[pl.ds(base, _BPW)], iip)
    pltpu.sync_copy(in_hbm.at[pl.ds(base, _BPW)], iin)
    pltpu.sync_copy(pp_hbm.at[pl.ds(base, _BPW)], ipp)
    pltpu.sync_copy(pn_hbm.at[pl.ds(base, _BPW)], ipn)
    c1 = pltpu.async_copy(x_hbm.at[iu], ru, sem)
    c2 = pltpu.async_copy(x_hbm.at[iip], rip, sem)
    c3 = pltpu.async_copy(x_hbm.at[iin], rin, sem)
    c4 = pltpu.async_copy(x_hbm.at[ipp], rpp, sem)
    c5 = pltpu.async_copy(x_hbm.at[ipn], rpn, sem)
    c1.wait(); c2.wait(); c3.wait(); c4.wait(); c5.wait()

    lane = lax.iota(jnp.int32, 16)

    def body(g, carry):
        rows = g * 16 + lane
        accp = jnp.zeros((16,), jnp.float32)
        accn = jnp.zeros((16,), jnp.float32)
        for d_ in range(D):
            cols = jnp.full((16,), d_, jnp.int32)
            ue = plsc.load_gather(ru, [rows, cols])
            ipe = plsc.load_gather(rip, [rows, cols])
            ine = plsc.load_gather(rin, [rows, cols])
            ppe = plsc.load_gather(rpp, [rows, cols])
            pne = plsc.load_gather(rpn, [rows, cols])
            accp = accp + ue * ipe + ue * ppe + ipe * ppe
            accn = accn + ue * ine + ue * pne + ine * pne
        op_v[pl.ds(g * 16, 16)] = accp
        on_v[pl.ds(g * 16, 16)] = accn
        return carry

    lax.fori_loop(0, _BPW // 16, body, 0)
    pltpu.sync_copy(op_v, outp_hbm.at[pl.ds(base, _BPW)])
    pltpu.sync_copy(on_v, outn_hbm.at[pl.ds(base, _BPW)])


def _decode(x, user, item_p, item_n, price_p, price_n):
    mesh = plsc.VectorSubcoreMesh(core_axis_name="c", subcore_axis_name="s")
    f = pl.kernel(
        _decode_body,
        mesh=mesh,
        compiler_params=pltpu.CompilerParams(needs_layout_passes=False),
        out_type=(
            jax.ShapeDtypeStruct((B,), jnp.float32),
            jax.ShapeDtypeStruct((B,), jnp.float32),
        ),
        scratch_types=[
            pltpu.VMEM((_BPW,), jnp.int32),
            pltpu.VMEM((_BPW,), jnp.int32),
            pltpu.VMEM((_BPW,), jnp.int32),
            pltpu.VMEM((_BPW,), jnp.int32),
            pltpu.VMEM((_BPW,), jnp.int32),
            pltpu.VMEM((_BPW, D), jnp.float32),
            pltpu.VMEM((_BPW, D), jnp.float32),
            pltpu.VMEM((_BPW, D), jnp.float32),
            pltpu.VMEM((_BPW, D), jnp.float32),
            pltpu.VMEM((_BPW, D), jnp.float32),
            pltpu.VMEM((_BPW,), jnp.float32),
            pltpu.VMEM((_BPW,), jnp.float32),
            pltpu.SemaphoreType.DMA,
        ],
    )
    return f(x, user, item_p, item_n, price_p, price_n)


@jax.jit
def kernel(feature, adj, user, item_p, item_n, price_p, price_n, W, b):
    x = _encode(feature, adj, W, b.reshape(1, D))
    i32 = jnp.int32
    return _decode(x, user.astype(i32), item_p.astype(i32),
                   item_n.astype(i32), price_p.astype(i32),
                   price_n.astype(i32))


# speedup vs baseline: 1.2549x; 1.2549x over previous
import functools

import jax
import jax.numpy as jnp
from jax import lax
from jax.experimental import pallas as pl
from jax.experimental.pallas import tpu as pltpu
from jax.experimental.pallas import tpu_sc as plsc

N = 10000
F = 128
D = 128
B = 4096

ROW_BLK = 400


def _encode_body(feat_ref, adj_ref, w_ref, b_ref, x_ref, support_ref):
    @pl.when(pl.program_id(0) == 0)
    def _():
        s = jnp.dot(
            feat_ref[...].astype(jnp.bfloat16),
            w_ref[...].astype(jnp.bfloat16),
            preferred_element_type=jnp.float32,
        )
        support_ref[...] = s.astype(jnp.bfloat16)

    acc = jnp.dot(
        adj_ref[...].astype(jnp.bfloat16),
        support_ref[...],
        preferred_element_type=jnp.float32,
    )
    x_ref[...] = jnp.tanh(acc + b_ref[...])


def _encode(feature, adj, W, b2d):
    grid = (N // ROW_BLK,)
    return pl.pallas_call(
        _encode_body,
        grid=grid,
        in_specs=[
            pl.BlockSpec((N, F), lambda i: (0, 0)),
            pl.BlockSpec((ROW_BLK, N), lambda i: (i, 0)),
            pl.BlockSpec((F, D), lambda i: (0, 0)),
            pl.BlockSpec((1, D), lambda i: (0, 0)),
        ],
        out_specs=pl.BlockSpec((ROW_BLK, D), lambda i: (i, 0)),
        out_shape=jax.ShapeDtypeStruct((N, D), jnp.float32),
        scratch_shapes=[pltpu.VMEM((N, D), jnp.bfloat16)],
    )(feature, adj, W, b2d)


_NC = 2
_NS = 16
_NW = _NC * _NS
_BPW = B // _NW


def _decode_body(x_hbm, u_hbm, ip_hbm, in_hbm, pp_hbm, pn_hbm,
                 outp_hbm, outn_hbm,
                 iu, iip, iin, ipp, ipn,
                 ru, rip, rin, rpp, rpn,
                 op_v, on_v, sem):
    wid = lax.axis_index("s") * _NC + lax.axis_index("c")
    base = wid * _BPW
    pltpu.sync_copy(u_hbm.at[pl.ds(base, _BPW)], iu)
    pltpu.sync_copy(ip_hbm.at[pl.ds(base, _BPW)], iip)
    pltpu.sync_copy(in_hbm.at[pl.ds(base, _BPW)], iin)
    pltpu.sync_copy(pp_hbm.at[pl.ds(base, _BPW)], ipp)
    pltpu.sync_copy(pn_hbm.at[pl.ds(base, _BPW)], ipn)
    c1 = pltpu.async_copy(x_hbm.at[iu], ru, sem)
    c2 = pltpu.async_copy(x_hbm.at[iip], rip, sem)
    c3 = pltpu.async_copy(x_hbm.at[iin], rin, sem)
    c4 = pltpu.async_copy(x_hbm.at[ipp], rpp, sem)
    c5 = pltpu.async_copy(x_hbm.at[ipn], rpn, sem)
    c1.wait(); c2.wait(); c3.wait(); c4.wait(); c5.wait()

    lane = lax.iota(jnp.int32, 16)

    def body(g, carry):
        accp_vec = jnp.zeros((16,), jnp.float32)
        accn_vec = jnp.zeros((16,), jnp.float32)
        for jj in range(16):
            j = g * 16 + jj
            accp = jnp.zeros((16,), jnp.float32)
            accn = jnp.zeros((16,), jnp.float32)
            for ci in range(D // 16):
                sl = pl.ds(ci * 16, 16)
                ue = ru[j, sl]
                ipe = rip[j, sl]
                ine = rin[j, sl]
                ppe = rpp[j, sl]
                pne = rpn[j, sl]
                accp = accp + ue * (ipe + ppe) + ipe * ppe
                accn = accn + ue * (ine + pne) + ine * pne
            accp_vec = jnp.where(lane == jj, jnp.sum(accp), accp_vec)
            accn_vec = jnp.where(lane == jj, jnp.sum(accn), accn_vec)
        op_v[pl.ds(g * 16, 16)] = accp_vec
        on_v[pl.ds(g * 16, 16)] = accn_vec
        return carry

    lax.fori_loop(0, _BPW // 16, body, 0)
    pltpu.sync_copy(op_v, outp_hbm.at[pl.ds(base, _BPW)])
    pltpu.sync_copy(on_v, outn_hbm.at[pl.ds(base, _BPW)])


def _decode(x, user, item_p, item_n, price_p, price_n):
    mesh = plsc.VectorSubcoreMesh(core_axis_name="c", subcore_axis_name="s")
    f = pl.kernel(
        _decode_body,
        mesh=mesh,
        compiler_params=pltpu.CompilerParams(needs_layout_passes=False),
        out_type=(
            jax.ShapeDtypeStruct((B,), jnp.float32),
            jax.ShapeDtypeStruct((B,), jnp.float32),
        ),
        scratch_types=[
            pltpu.VMEM((_BPW,), jnp.int32),
            pltpu.VMEM((_BPW,), jnp.int32),
            pltpu.VMEM((_BPW,), jnp.int32),
            pltpu.VMEM((_BPW,), jnp.int32),
            pltpu.VMEM((_BPW,), jnp.int32),
            pltpu.VMEM((_BPW, D), jnp.float32),
            pltpu.VMEM((_BPW, D), jnp.float32),
            pltpu.VMEM((_BPW, D), jnp.float32),
            pltpu.VMEM((_BPW, D), jnp.float32),
            pltpu.VMEM((_BPW, D), jnp.float32),
            pltpu.VMEM((_BPW,), jnp.float32),
            pltpu.VMEM((_BPW,), jnp.float32),
            pltpu.SemaphoreType.DMA,
        ],
    )
    return f(x, user, item_p, item_n, price_p, price_n)


@jax.jit
def kernel(feature, adj, user, item_p, item_n, price_p, price_n, W, b):
    x = _encode(feature, adj, W, b.reshape(1, D))
    i32 = jnp.int32
    return _decode(x, user.astype(i32), item_p.astype(i32),
                   item_n.astype(i32), price_p.astype(i32),
                   price_n.astype(i32))
